# Initial kernel scaffold; baseline (speedup 1.0000x reference)
import jax
import jax.numpy as jnp
from jax.experimental import pallas as pl


def kernel(x, edge_index, W_neigh, W_self):
    raise NotImplementedError("write your pallas kernel here")


# speedup vs baseline: 7.3735x; 7.3735x over previous
import functools

import jax
import jax.numpy as jnp
from jax import lax
from jax.experimental import pallas as pl
from jax.experimental.pallas import tpu as pltpu
from jax.experimental.pallas import tpu_sc as plsc

LANES = 128
NCORES = 2
NSUB = 16
NW = NCORES * NSUB


def _sc_aggregate_body(n_acc, rows_per_w, x_hbm, src_hbm, dst_hbm,
                       partial_hbm, deg_hbm,
                       acc_sh, deg_sh, src_v, dst_v, rows_v, ones_v, zvec_v,
                       sem):
    cid = lax.axis_index("c")
    sid = lax.axis_index("s")
    wid = cid * NSUB + sid
    rows_per_tile = n_acc // NSUB
    tbase = sid * rows_per_tile

    def zrow(r, _):
        for c in range(8):
            rows_v[r, pl.ds(c * 16, 16)] = jnp.zeros((16,), jnp.float32)
        return _
    lax.fori_loop(0, LANES, zrow, 0)

    def zvec(i, _):
        zvec_v[pl.ds(i * 16, 16)] = jnp.zeros((16,), jnp.float32)
        return _
    lax.fori_loop(0, rows_per_tile // 16, zvec, 0)

    def ovec(i, _):
        ones_v[pl.ds(i * 16, 16)] = jnp.ones((16,), jnp.float32)
        return _
    lax.fori_loop(0, LANES // 16, ovec, 0)

    for k in range(rows_per_tile // LANES):
        pltpu.sync_copy(rows_v, acc_sh.at[pl.ds(tbase + k * LANES, LANES)])
    pltpu.sync_copy(zvec_v, deg_sh.at[pl.ds(tbase, rows_per_tile)])
    plsc.subcore_barrier()

    def edge_chunk(j, _):
        row = wid * rows_per_w + j
        pltpu.sync_copy(src_hbm.at[row], src_v)
        pltpu.sync_copy(dst_hbm.at[row], dst_v)
        pltpu.async_copy(x_hbm.at[src_v], rows_v, sem).wait()
        pltpu.sync_copy(rows_v, acc_sh.at[dst_v], add=True)
        pltpu.sync_copy(ones_v, deg_sh.at[dst_v], add=True)
        return _
    lax.fori_loop(0, rows_per_w, edge_chunk, 0)
    plsc.subcore_barrier()

    obase = cid * n_acc + tbase
    for k in range(rows_per_tile // LANES):
        pltpu.sync_copy(acc_sh.at[pl.ds(tbase + k * LANES, LANES)],
                        partial_hbm.at[pl.ds(obase + k * LANES, LANES)])
    pltpu.sync_copy(deg_sh.at[pl.ds(tbase, rows_per_tile)],
                    deg_hbm.at[pl.ds(obase, rows_per_tile)])


def _sc_aggregate(x, src2d, dst2d, n_acc, rows_per_w):
    d = x.shape[1]
    rows_per_tile = n_acc // NSUB
    mesh = plsc.VectorSubcoreMesh(core_axis_name="c", subcore_axis_name="s")
    body = functools.partial(_sc_aggregate_body, n_acc, rows_per_w)
    return pl.kernel(
        body,
        out_type=(
            jax.ShapeDtypeStruct((NCORES * n_acc, d), jnp.float32),
            jax.ShapeDtypeStruct((NCORES * n_acc,), jnp.float32),
        ),
        mesh=mesh,
        scratch_types=[
            pltpu.VMEM_SHARED((n_acc, d), jnp.float32),
            pltpu.VMEM_SHARED((n_acc,), jnp.float32),
            pltpu.VMEM((LANES,), jnp.int32),
            pltpu.VMEM((LANES,), jnp.int32),
            pltpu.VMEM((LANES, d), jnp.float32),
            pltpu.VMEM((LANES,), jnp.float32),
            pltpu.VMEM((rows_per_tile,), jnp.float32),
            pltpu.SemaphoreType.DMA,
        ],
    )(x, src2d, dst2d)


def _tc_update_body(nblk, p0, p1, deg, x, wn, ws, out):
    br = out.shape[0]
    msg = p0[...] + p1[...]
    d = deg[0:1, :] + deg[1:2, :]
    r = 1.0 / jnp.maximum(d, 1.0)
    ii = lax.broadcasted_iota(jnp.int32, (br, br), 0)
    jj = lax.broadcasted_iota(jnp.int32, (br, br), 1)
    diag = jnp.where(ii == jj, jnp.broadcast_to(r, (br, br)), 0.0)
    rcol = jnp.sum(diag, axis=1, keepdims=True)
    h = msg * rcol
    out[...] = (jnp.dot(h, wn[...], preferred_element_type=jnp.float32)
                + jnp.dot(x[...], ws[...], preferred_element_type=jnp.float32))


def _tc_update(partial, deg2, x, w_neigh, w_self, n_acc):
    n, d = x.shape
    br = 256
    nblk = n_acc // br
    return pl.pallas_call(
        functools.partial(_tc_update_body, nblk),
        grid=(nblk,),
        in_specs=[
            pl.BlockSpec((br, d), lambda i: (i, 0)),
            pl.BlockSpec((br, d), lambda i: (i + nblk, 0)),
            pl.BlockSpec((NCORES, br), lambda i: (0, i)),
            pl.BlockSpec((br, d), lambda i: (i, 0)),
            pl.BlockSpec((d, d), lambda i: (0, 0)),
            pl.BlockSpec((d, d), lambda i: (0, 0)),
        ],
        out_specs=pl.BlockSpec((br, d), lambda i: (i, 0)),
        out_shape=jax.ShapeDtypeStruct((n, d), jnp.float32),
    )(partial, partial, deg2, x, w_neigh, w_self)


def kernel(x, edge_index, W_neigh, W_self):
    n, d = x.shape
    e = edge_index.shape[1]
    n_acc = ((n + 1 + NSUB * LANES - 1) // (NSUB * LANES)) * (NSUB * LANES)
    rows_per_w = (e + NW * LANES - 1) // (NW * LANES)
    e_pad = NW * rows_per_w * LANES

    src = edge_index[0].astype(jnp.int32)
    dst = edge_index[1].astype(jnp.int32)
    padi = jnp.arange(e_pad - e, dtype=jnp.int32)
    src_p = jnp.concatenate([src, padi % n])
    dst_p = jnp.concatenate([dst, n + padi % (n_acc - n)])
    src2d = src_p.reshape(NW * rows_per_w, LANES)
    dst2d = dst_p.reshape(NW * rows_per_w, LANES)

    partial, deg = _sc_aggregate(x, src2d, dst2d, n_acc, rows_per_w)
    return _tc_update(partial, deg.reshape(NCORES, n_acc), x,
                      W_neigh, W_self, n_acc)


# speedup vs baseline: 13.4767x; 1.8277x over previous
import functools

import jax
import jax.numpy as jnp
from jax import lax
from jax.experimental import pallas as pl
from jax.experimental.pallas import tpu as pltpu
from jax.experimental.pallas import tpu_sc as plsc

LANES = 128
NCORES = 2
NSUB = 16
NW = NCORES * NSUB
GRP = 4


def _sc_aggregate_body(n_acc, n_r, x_hbm, src_hbm, dst_hbm,
                       partial_hbm, deg_hbm,
                       acc_sh, deg_sh, src_v, dst_v, rows_v, ones_v, zvec_v,
                       sem_g, sem_s, sem_d, sem_i):
    cid = lax.axis_index("c")
    sid = lax.axis_index("s")
    wid = cid * NSUB + sid
    rows_per_tile = n_acc // NSUB
    tbase = sid * rows_per_tile
    ngrp = n_r // GRP

    def zrow(r, _):
        for c in range(8):
            rows_v[0, r, pl.ds(c * 16, 16)] = jnp.zeros((16,), jnp.float32)
        return _
    lax.fori_loop(0, LANES, zrow, 0)

    def zvec(i, _):
        zvec_v[pl.ds(i * 16, 16)] = jnp.zeros((16,), jnp.float32)
        return _
    lax.fori_loop(0, rows_per_tile // 16, zvec, 0)

    def ovec(i, _):
        ones_v[pl.ds(i * 16, 16)] = jnp.ones((16,), jnp.float32)
        return _
    lax.fori_loop(0, LANES // 16, ovec, 0)

    for k in range(rows_per_tile // LANES):
        pltpu.sync_copy(rows_v.at[0], acc_sh.at[pl.ds(tbase + k * LANES, LANES)])
    pltpu.sync_copy(zvec_v, deg_sh.at[pl.ds(tbase, rows_per_tile)])
    plsc.subcore_barrier()

    def idx_start(g, ib):
        pltpu.async_copy(src_hbm.at[pl.ds((wid * ngrp + g) * GRP, GRP)],
                         src_v.at[ib], sem_i.at[0])
        pltpu.async_copy(dst_hbm.at[pl.ds((wid * ngrp + g) * GRP, GRP)],
                         dst_v.at[ib], sem_i.at[1])

    def idx_wait(g, ib):
        pltpu.make_async_copy(src_hbm.at[pl.ds((wid * ngrp + g) * GRP, GRP)],
                              src_v.at[ib], sem_i.at[0]).wait()
        pltpu.make_async_copy(dst_hbm.at[pl.ds((wid * ngrp + g) * GRP, GRP)],
                              dst_v.at[ib], sem_i.at[1]).wait()

    def gather_start(ib, t, b):
        pltpu.async_copy(x_hbm.at[src_v.at[ib, t]], rows_v.at[b], sem_g.at[b])

    def gather_wait(ib, t, b):
        pltpu.make_async_copy(x_hbm.at[src_v.at[ib, t]], rows_v.at[b],
                              sem_g.at[b]).wait()

    def scatter_start(ib, t, b):
        pltpu.async_copy(rows_v.at[b], acc_sh.at[dst_v.at[ib, t]],
                         sem_s.at[b], add=True)
        pltpu.async_copy(ones_v, deg_sh.at[dst_v.at[ib, t]], sem_d.at[b],
                         add=True)

    def scatter_wait(ib, t, b):
        pltpu.make_async_copy(rows_v.at[b], acc_sh.at[dst_v.at[ib, t]],
                              sem_s.at[b]).wait()
        pltpu.make_async_copy(ones_v, deg_sh.at[dst_v.at[ib, t]],
                              sem_d.at[b]).wait()

    idx_start(0, 0)
    idx_wait(0, 0)
    if ngrp > 1:
        idx_start(1, 1)
    gather_start(0, 0, 0)
    gather_start(0, 1, 1)

    def outer(g, carry):
        gb = lax.rem(g, 2)
        for t in range(GRP):
            b = t % 2
            if t == 0:
                @pl.when(g + 1 < ngrp)
                def _wait_next_idx():
                    idx_wait(g + 1, 1 - gb)
            gather_wait(gb, t, b)
            scatter_start(gb, t, b)
            scatter_wait(gb, t, b)
            if t < GRP - 2:
                gather_start(gb, t + 2, b)
            else:
                @pl.when(g + 1 < ngrp)
                def _prefetch_rows(t=t, b=b):
                    gather_start(1 - gb, t + 2 - GRP, b)
            if t == GRP - 1:
                @pl.when(g + 2 < ngrp)
                def _prefetch_idx():
                    idx_start(g + 2, gb)
        return carry
    lax.fori_loop(0, ngrp, outer, 0)
    plsc.subcore_barrier()

    obase = cid * n_acc + tbase
    for k in range(rows_per_tile // LANES):
        pltpu.sync_copy(acc_sh.at[pl.ds(tbase + k * LANES, LANES)],
                        partial_hbm.at[pl.ds(obase + k * LANES, LANES)])
    pltpu.sync_copy(deg_sh.at[pl.ds(tbase, rows_per_tile)],
                    deg_hbm.at[pl.ds(obase, rows_per_tile)])


def _sc_aggregate(x, src2d, dst2d, n_acc, rows_per_w):
    d = x.shape[1]
    rows_per_tile = n_acc // NSUB
    mesh = plsc.VectorSubcoreMesh(core_axis_name="c", subcore_axis_name="s")
    body = functools.partial(_sc_aggregate_body, n_acc, rows_per_w)
    return pl.kernel(
        body,
        out_type=(
            jax.ShapeDtypeStruct((NCORES * n_acc, d), jnp.float32),
            jax.ShapeDtypeStruct((NCORES * n_acc,), jnp.float32),
        ),
        mesh=mesh,
        scratch_types=[
            pltpu.VMEM_SHARED((n_acc, d), jnp.float32),
            pltpu.VMEM_SHARED((n_acc,), jnp.float32),
            pltpu.VMEM((2, GRP, LANES), jnp.int32),
            pltpu.VMEM((2, GRP, LANES), jnp.int32),
            pltpu.VMEM((2, LANES, d), jnp.float32),
            pltpu.VMEM((LANES,), jnp.float32),
            pltpu.VMEM((rows_per_tile,), jnp.float32),
            pltpu.SemaphoreType.DMA((2,)),
            pltpu.SemaphoreType.DMA((2,)),
            pltpu.SemaphoreType.DMA((2,)),
            pltpu.SemaphoreType.DMA((2,)),
        ],
    )(x, src2d, dst2d)


def _tc_update_body(nblk, p0, p1, deg, x, wn, ws, out):
    br = out.shape[0]
    msg = p0[...] + p1[...]
    d = deg[0:1, :] + deg[1:2, :]
    r = 1.0 / jnp.maximum(d, 1.0)
    ii = lax.broadcasted_iota(jnp.int32, (br, br), 0)
    jj = lax.broadcasted_iota(jnp.int32, (br, br), 1)
    diag = jnp.where(ii == jj, jnp.broadcast_to(r, (br, br)), 0.0)
    rcol = jnp.sum(diag, axis=1, keepdims=True)
    h = msg * rcol
    out[...] = (jnp.dot(h, wn[...], preferred_element_type=jnp.float32)
                + jnp.dot(x[...], ws[...], preferred_element_type=jnp.float32))


def _tc_update(partial, deg2, x, w_neigh, w_self, n_acc):
    n, d = x.shape
    br = 256
    nblk = n_acc // br
    return pl.pallas_call(
        functools.partial(_tc_update_body, nblk),
        grid=(nblk,),
        in_specs=[
            pl.BlockSpec((br, d), lambda i: (i, 0)),
            pl.BlockSpec((br, d), lambda i: (i + nblk, 0)),
            pl.BlockSpec((NCORES, br), lambda i: (0, i)),
            pl.BlockSpec((br, d), lambda i: (i, 0)),
            pl.BlockSpec((d, d), lambda i: (0, 0)),
            pl.BlockSpec((d, d), lambda i: (0, 0)),
        ],
        out_specs=pl.BlockSpec((br, d), lambda i: (i, 0)),
        out_shape=jax.ShapeDtypeStruct((n, d), jnp.float32),
    )(partial, partial, deg2, x, w_neigh, w_self)


def kernel(x, edge_index, W_neigh, W_self):
    n, d = x.shape
    e = edge_index.shape[1]
    n_acc = ((n + 1 + NSUB * LANES - 1) // (NSUB * LANES)) * (NSUB * LANES)
    rows_per_w = (e + NW * LANES - 1) // (NW * LANES)
    rows_per_w = ((rows_per_w + GRP - 1) // GRP) * GRP
    e_pad = NW * rows_per_w * LANES

    src = edge_index[0].astype(jnp.int32)
    dst = edge_index[1].astype(jnp.int32)
    padi = jnp.arange(e_pad - e, dtype=jnp.int32)
    src_p = jnp.concatenate([src, padi % n])
    dst_p = jnp.concatenate([dst, n + padi % (n_acc - n)])
    src2d = src_p.reshape(NW * rows_per_w, LANES)
    dst2d = dst_p.reshape(NW * rows_per_w, LANES)

    partial, deg = _sc_aggregate(x, src2d, dst2d, n_acc, rows_per_w)
    return _tc_update(partial, deg.reshape(NCORES, n_acc), x,
                      W_neigh, W_self, n_acc)
